# Initial kernel scaffold; baseline (speedup 1.0000x reference)
import jax
import jax.numpy as jnp
from jax.experimental import pallas as pl


def kernel(nodes, history_uv, history_r, u2e, v2e, r2e, w_r1_w, w_r1_b, w_r2_w, w_r2_b, att1_w, att1_b, att2_w, att2_b, att3_w, att3_b, lin1_w, lin1_b):
    raise NotImplementedError("write your pallas kernel here")


# speedup vs baseline: 7.6668x; 7.6668x over previous
import functools
import jax
import jax.numpy as jnp
from jax import lax
from jax.experimental import pallas as pl
from jax.experimental.pallas import tpu as pltpu
from jax.experimental.pallas import tpu_sc as plsc

B = 16384
H = 50
D = 64
NV = 100000
NR = 5

NC = 2
NS = 16
NW = NC * NS
NODES_PER_W = B // NW
IDX_PER_W = NODES_PER_W * H
GCHUNK = 1600
NCHUNK = IDX_PER_W // GCHUNK

_mesh = plsc.VectorSubcoreMesh(core_axis_name="c", subcore_axis_name="s")


def _prep_body(v_ref, w_ref, o_ref):
    o_ref[...] = jnp.dot(v_ref[...], w_ref[...], preferred_element_type=jnp.float32)


def _prep(v2e, wa_t):
    blk = 5000
    return pl.pallas_call(
        _prep_body,
        grid=(NV // blk,),
        in_specs=[
            pl.BlockSpec((blk, D), lambda i: (i, 0)),
            pl.BlockSpec((D, D), lambda i: (0, 0)),
        ],
        out_specs=pl.BlockSpec((blk, D), lambda i: (i, 0)),
        out_shape=jax.ShapeDtypeStruct((NV, D), jnp.float32),
    )(v2e, wa_t)


def _pad_body(huv_ref, hrr_ref, o_ref):
    blk = huv_ref.shape[0]
    z = jnp.zeros((blk, 128 - 2 * H), dtype=jnp.int32)
    o_ref[...] = jnp.concatenate(
        [huv_ref[...], z[:, : 64 - H], hrr_ref[...], z[:, : 64 - H]], axis=1)


def _pad_hist(history_uv, history_r):
    blk = 2000
    return pl.pallas_call(
        _pad_body,
        grid=(NV // blk,),
        in_specs=[
            pl.BlockSpec((blk, H), lambda i: (i, 0)),
            pl.BlockSpec((blk, H), lambda i: (i, 0)),
        ],
        out_specs=pl.BlockSpec((blk, 128), lambda i: (i, 0)),
        out_shape=jax.ShapeDtypeStruct((NV, 128), jnp.int32),
    )(history_uv, history_r)


@functools.partial(
    pl.kernel,
    mesh=_mesh,
    compiler_params=pltpu.CompilerParams(use_tc_tiling_on_sc=False),
    out_type=(
        jax.ShapeDtypeStruct((B, 128), jnp.int32),
        jax.ShapeDtypeStruct((B, D), jnp.float32),
    ),
    scratch_types=[
        pltpu.VMEM((NODES_PER_W,), jnp.int32),
        pltpu.VMEM((NODES_PER_W, 128), jnp.int32),
        pltpu.VMEM((NODES_PER_W, D), jnp.float32),
        pltpu.SemaphoreType.DMA,
    ],
)
def _sc_gather_nodes(nodes_hbm, hist_hbm, u2e_hbm,
                     hist_out, uv_out,
                     idx_v, hist_v, uv_v, sem):
    wid = lax.axis_index("s") * NC + lax.axis_index("c")
    base = wid * NODES_PER_W
    pltpu.sync_copy(nodes_hbm.at[pl.ds(base, NODES_PER_W)], idx_v)
    a = pltpu.async_copy(hist_hbm.at[idx_v], hist_v, sem)
    b = pltpu.async_copy(u2e_hbm.at[idx_v], uv_v, sem)
    a.wait()
    b.wait()
    pltpu.sync_copy(hist_v, hist_out.at[pl.ds(base, NODES_PER_W)])
    pltpu.sync_copy(uv_v, uv_out.at[pl.ds(base, NODES_PER_W)])


@functools.partial(
    pl.kernel,
    mesh=_mesh,
    compiler_params=pltpu.CompilerParams(use_tc_tiling_on_sc=False),
    out_type=jax.ShapeDtypeStruct((B * H, D), jnp.float32),
    scratch_types=[
        pltpu.VMEM((GCHUNK,), jnp.int32),
        pltpu.VMEM((GCHUNK, D), jnp.float32),
        pltpu.SemaphoreType.DMA,
    ],
)
def _sc_gather_items(vtab_hbm, idx_hbm, out_hbm, idx_v, rows_v, sem):
    wid = lax.axis_index("s") * NC + lax.axis_index("c")

    def body(c, carry):
        base = wid * IDX_PER_W + c * GCHUNK
        pltpu.sync_copy(idx_hbm.at[pl.ds(base, GCHUNK)], idx_v)
        pltpu.async_copy(vtab_hbm.at[idx_v], rows_v, sem).wait()
        pltpu.sync_copy(rows_v, out_hbm.at[pl.ds(base, GCHUNK)])
        return carry

    lax.fori_loop(0, NCHUNK, body, 0)


def _main_body(vg_ref, hr_ref, uv_ref, r2e_ref, w1b_t_ref, w_r1_b_ref,
               w2_t_ref, w_r2_b_ref, a1o_t_ref, a1u_t_ref, att1_b_ref,
               a2_t_ref, att2_b_ref, att3_ref, l1u_t_ref, l1n_t_ref,
               lin1_b_ref, o_ref, *, nblk):
    nh = nblk * H
    vg = vg_ref[...].reshape(nh, D)
    hr = hr_ref[...]
    uv = uv_ref[...]

    rproj = jnp.dot(r2e_ref[...], w1b_t_ref[...],
                    preferred_element_type=jnp.float32) + w_r1_b_ref[...]
    onehot = (hr[:, :, None] ==
              lax.broadcasted_iota(jnp.int32, (H, nblk, NR), 2)
              ).astype(jnp.float32).reshape(nh, NR)
    x1 = jnp.maximum(vg + jnp.dot(onehot, rproj,
                                  preferred_element_type=jnp.float32), 0.0)
    o = jnp.maximum(jnp.dot(x1, w2_t_ref[...],
                            preferred_element_type=jnp.float32)
                    + w_r2_b_ref[...], 0.0)

    up = jnp.dot(uv, a1u_t_ref[...], preferred_element_type=jnp.float32)
    upb = jnp.broadcast_to(up[None, :, :], (H, nblk, D)).reshape(nh, D)
    a1 = jnp.maximum(jnp.dot(o, a1o_t_ref[...],
                             preferred_element_type=jnp.float32)
                     + upb + att1_b_ref[...], 0.0)
    a2 = jnp.maximum(jnp.dot(a1, a2_t_ref[...],
                             preferred_element_type=jnp.float32)
                     + att2_b_ref[...], 0.0)

    logits = jnp.sum(a2.reshape(H, nblk, D) * att3_ref[...], axis=2)
    m = jnp.max(logits, axis=0, keepdims=True)
    e = jnp.exp(logits - m)
    w = e / jnp.sum(e, axis=0, keepdims=True)

    o3 = o.reshape(H, nblk, D)
    neigh = jnp.sum(o3 * w[:, :, None], axis=0)

    out = jnp.maximum(jnp.dot(uv, l1u_t_ref[...],
                              preferred_element_type=jnp.float32)
                      + jnp.dot(neigh, l1n_t_ref[...],
                                preferred_element_type=jnp.float32)
                      + lin1_b_ref[...], 0.0)
    o_ref[...] = out


def _main(vg3, hr_t, uv, r2e, w1b_t, w_r1_b, w2_t, w_r2_b,
          a1o_t, a1u_t, att1_b, a2_t, att2_b, att3, l1u_t, l1n_t, lin1_b):
    nblk = 256
    grid = (B // nblk,)
    full = lambda shape: pl.BlockSpec(shape, lambda i: tuple(0 for _ in shape))
    return pl.pallas_call(
        functools.partial(_main_body, nblk=nblk),
        grid=grid,
        in_specs=[
            pl.BlockSpec((H, nblk, D), lambda i: (0, i, 0)),
            pl.BlockSpec((H, nblk), lambda i: (0, i)),
            pl.BlockSpec((nblk, D), lambda i: (i, 0)),
            full((NR, D)), full((D, D)), full((D,)),
            full((D, D)), full((D,)),
            full((D, D)), full((D, D)), full((D,)),
            full((D, D)), full((D,)),
            full((D,)),
            full((D, D)), full((D, D)), full((D,)),
        ],
        out_specs=pl.BlockSpec((nblk, D), lambda i: (i, 0)),
        out_shape=jax.ShapeDtypeStruct((B, D), jnp.float32),
    )(vg3, hr_t, uv, r2e, w1b_t, w_r1_b, w2_t, w_r2_b,
      a1o_t, a1u_t, att1_b, a2_t, att2_b, att3, l1u_t, l1n_t, lin1_b)


def kernel(nodes, history_uv, history_r, u2e, v2e, r2e,
           w_r1_w, w_r1_b, w_r2_w, w_r2_b,
           att1_w, att1_b, att2_w, att2_b, att3_w, att3_b,
           lin1_w, lin1_b):
    wa_t = jnp.transpose(w_r1_w[:, :D])
    w1b_t = jnp.transpose(w_r1_w[:, D:])
    w2_t = jnp.transpose(w_r2_w)
    a1o_t = jnp.transpose(att1_w[:, :D])
    a1u_t = jnp.transpose(att1_w[:, D:])
    a2_t = jnp.transpose(att2_w)
    att3 = att3_w[0]
    l1u_t = jnp.transpose(lin1_w[:, :D])
    l1n_t = jnp.transpose(lin1_w[:, D:])

    vproj = _prep(v2e, wa_t)
    hist_pad = _pad_hist(history_uv, history_r)
    hist_g, uv = _sc_gather_nodes(nodes, hist_pad, u2e)
    hi = hist_g[:, :H]
    hr = hist_g[:, 64:64 + H]
    idx = jnp.transpose(hi).reshape(-1)
    vg = _sc_gather_items(vproj, idx)
    vg3 = vg.reshape(H, B, D)
    hr_t = jnp.transpose(hr)
    return _main(vg3, hr_t, uv, r2e, w1b_t, w_r1_b, w2_t, w_r2_b,
                 a1o_t, a1u_t, att1_b, a2_t, att2_b, att3, l1u_t, l1n_t, lin1_b)
